# baseline (speedup 1.0000x reference)
import jax
import jax.numpy as jnp
from jax.experimental import pallas as pl

_NCHUNK = 4
_BSPLIT = 2


def _update_kernel(idx_ref, x_ref, w_ref, b_ref, m_ref, k_ref, v_ref, o_ref):
    n0 = pl.program_id(0) * _NCHUNK
    x = x_ref[...]
    idx = idx_ref[...]
    for c in range(_NCHUNK):
        w = w_ref[c]
        logits = jax.lax.dot_general(
            x, w, (((1,), (1,)), ((), ())),
            preferred_element_type=jnp.float32)
        alpha = jax.nn.sigmoid(logits + b_ref[c, 0][None, :])
        active = jnp.any(idx == n0 + c, axis=1)
        m = m_ref[:, c]
        k = k_ref[:, c, 0]
        v = v_ref[:, c, 0]
        upd = m * alpha[:, :, None] + k[:, :, None] * v[:, None, :]
        o_ref[:, c] = jnp.where(active[:, None, None], upd, m)


def kernel(M, M_k, M_v, indices_update, x_t, W, b):
    B, N, D, _ = M.shape
    input_dim = x_t.shape[1]
    Bc = B // _BSPLIT
    idx = indices_update.astype(jnp.int32)
    W3 = W.reshape(N, D, input_dim)
    b3 = b.reshape(N, 1, D)
    Mk4 = M_k.reshape(B, N, 1, D)
    Mv4 = M_v.reshape(B, N, 1, D)

    return pl.pallas_call(
        _update_kernel,
        grid=(N // _NCHUNK, _BSPLIT),
        in_specs=[
            pl.BlockSpec((Bc, idx.shape[1]), lambda n, i: (i, 0)),
            pl.BlockSpec((Bc, input_dim), lambda n, i: (i, 0)),
            pl.BlockSpec((_NCHUNK, D, input_dim), lambda n, i: (n, 0, 0)),
            pl.BlockSpec((_NCHUNK, 1, D), lambda n, i: (n, 0, 0)),
            pl.BlockSpec((Bc, _NCHUNK, D, D), lambda n, i: (i, n, 0, 0)),
            pl.BlockSpec((Bc, _NCHUNK, 1, D), lambda n, i: (i, n, 0, 0)),
            pl.BlockSpec((Bc, _NCHUNK, 1, D), lambda n, i: (i, n, 0, 0)),
        ],
        out_specs=pl.BlockSpec((Bc, _NCHUNK, D, D), lambda n, i: (i, n, 0, 0)),
        out_shape=jax.ShapeDtypeStruct((B, N, D, D), M.dtype),
    )(idx, x_t, W3, b3, M, Mk4, Mv4)


# speedup vs baseline: 1.2167x; 1.2167x over previous
import jax
import jax.numpy as jnp
from jax.experimental import pallas as pl

_NCHUNK = 4


def _update_kernel(idx_ref, x_ref, w_ref, b_ref, m_ref, k_ref, v_ref, o_ref):
    n0 = pl.program_id(0) * _NCHUNK
    x = x_ref[...]
    idx = idx_ref[...]
    for c in range(_NCHUNK):
        w = w_ref[c]
        logits = jax.lax.dot_general(
            x, w, (((1,), (1,)), ((), ())),
            preferred_element_type=jnp.float32)
        alpha = jax.nn.sigmoid(logits + b_ref[c, 0][None, :])
        active = jnp.any(idx == n0 + c, axis=1)
        m = m_ref[:, c]
        k = k_ref[:, c, 0]
        v = v_ref[:, c, 0]
        upd = m * alpha[:, :, None] + k[:, :, None] * v[:, None, :]
        o_ref[:, c] = jnp.where(active[:, None, None], upd, m)


def kernel(M, M_k, M_v, indices_update, x_t, W, b):
    B, N, D, _ = M.shape
    input_dim = x_t.shape[1]
    idx = indices_update.astype(jnp.int32)
    W3 = W.reshape(N, D, input_dim)
    b3 = b.reshape(N, 1, D)
    Mk4 = M_k.reshape(B, N, 1, D)
    Mv4 = M_v.reshape(B, N, 1, D)

    return pl.pallas_call(
        _update_kernel,
        grid=(N // _NCHUNK,),
        in_specs=[
            pl.BlockSpec(idx.shape, lambda n: (0, 0)),
            pl.BlockSpec((B, input_dim), lambda n: (0, 0)),
            pl.BlockSpec((_NCHUNK, D, input_dim), lambda n: (n, 0, 0)),
            pl.BlockSpec((_NCHUNK, 1, D), lambda n: (n, 0, 0)),
            pl.BlockSpec((B, _NCHUNK, D, D), lambda n: (0, n, 0, 0)),
            pl.BlockSpec((B, _NCHUNK, 1, D), lambda n: (0, n, 0, 0)),
            pl.BlockSpec((B, _NCHUNK, 1, D), lambda n: (0, n, 0, 0)),
        ],
        out_specs=pl.BlockSpec((B, _NCHUNK, D, D), lambda n: (0, n, 0, 0)),
        out_shape=jax.ShapeDtypeStruct((B, N, D, D), M.dtype),
    )(idx, x_t, W3, b3, M, Mk4, Mv4)
